# Initial kernel scaffold; baseline (speedup 1.0000x reference)
import jax
import jax.numpy as jnp
from jax.experimental import pallas as pl


def kernel(x, edge_index, W1, b1, W2, b2):
    raise NotImplementedError("write your pallas kernel here")


# speedup vs baseline: 8.8718x; 8.8718x over previous
import functools

import jax
import jax.numpy as jnp
from jax import lax
from jax.experimental import pallas as pl
from jax.experimental.pallas import tpu as pltpu
from jax.experimental.pallas import tpu_sc as plsc

N = 10000
D = 256
HALF = D // 2
NP = 10240
ROWS_PER_TILE = NP // 16
B = 128


def _zero_vec_rows(ref, nrows):
    z = jnp.zeros((16,), jnp.float32)

    def row(i, c):
        for k in range(8):
            ref[i, pl.ds(k * 16, 16)] = z
        return c

    lax.fori_loop(0, nrows, row, 0)


def _deg_body(dst_hbm, out0_hbm, out1_hbm, dst_v, ones_v, zero_v, acc):
    c = lax.axis_index("c")
    s = lax.axis_index("s")
    steps = dst_v.shape[0]

    one = jnp.ones((16,), jnp.float32)
    z = jnp.zeros((16,), jnp.float32)
    for k in range(8):
        ones_v[pl.ds(k * 16, 16)] = one
        zero_v[pl.ds(k * 16, 16)] = z

    for k in range(ROWS_PER_TILE // 128):
        pltpu.sync_copy(zero_v, acc.at[pl.ds(s * ROWS_PER_TILE + k * 128, 128)])
    plsc.subcore_barrier()

    pltpu.sync_copy(dst_hbm.at[c * 16 + s], dst_v)

    def step(j, carry):
        pltpu.sync_copy(ones_v, acc.at[dst_v.at[j]], add=True)
        return carry

    lax.fori_loop(0, steps, step, 0)
    plsc.subcore_barrier()

    def copy_out(out_hbm):
        pltpu.sync_copy(acc.at[pl.ds(s * ROWS_PER_TILE, ROWS_PER_TILE)],
                        out_hbm.at[pl.ds(s * ROWS_PER_TILE, ROWS_PER_TILE)])

    @pl.when(c == 0)
    def _():
        copy_out(out0_hbm)

    @pl.when(c == 1)
    def _():
        copy_out(out1_hbm)


def _make_deg_kernel(steps):
    mesh = plsc.VectorSubcoreMesh(core_axis_name="c", subcore_axis_name="s")
    return pl.kernel(
        _deg_body,
        out_type=(jax.ShapeDtypeStruct((NP,), jnp.float32),
                  jax.ShapeDtypeStruct((NP,), jnp.float32)),
        mesh=mesh,
        scratch_types=[
            pltpu.VMEM((steps, B), jnp.int32),
            pltpu.VMEM((B,), jnp.float32),
            pltpu.VMEM((B,), jnp.float32),
            pltpu.VMEM_SHARED((NP,), jnp.float32),
        ],
    )


def _agg_body(src_hbm, dst_hbm, g0_hbm, g1_hbm, out0_hbm, out1_hbm,
              src_v, dst_v, buf0, buf1, acc, sem0, sem1):
    c = lax.axis_index("c")
    s = lax.axis_index("s")
    steps = src_v.shape[0]

    _zero_vec_rows(buf0, B)
    for k in range(ROWS_PER_TILE // 128):
        pltpu.sync_copy(buf0, acc.at[pl.ds(s * ROWS_PER_TILE + k * 128, 128)])
    plsc.subcore_barrier()

    def run(g_hbm, out_hbm):
        for phase in range(2):
            pltpu.sync_copy(src_hbm.at[phase * 16 + s], src_v)
            pltpu.sync_copy(dst_hbm.at[phase * 16 + s], dst_v)

            pltpu.async_copy(g_hbm.at[src_v.at[0]], buf0, sem0)

            def pair(t, carry):
                j0 = 2 * t

                @pl.when(j0 + 1 < steps)
                def _():
                    pltpu.async_copy(g_hbm.at[src_v.at[j0 + 1]], buf1, sem1)

                pltpu.make_async_copy(g_hbm.at[src_v.at[j0]], buf0, sem0).wait()
                pltpu.sync_copy(buf0, acc.at[dst_v.at[j0]], add=True)

                @pl.when(j0 + 2 < steps)
                def _():
                    pltpu.async_copy(g_hbm.at[src_v.at[j0 + 2]], buf0, sem0)

                @pl.when(j0 + 1 < steps)
                def _():
                    pltpu.make_async_copy(g_hbm.at[src_v.at[j0 + 1]], buf1,
                                          sem1).wait()
                    pltpu.sync_copy(buf1, acc.at[dst_v.at[j0 + 1]], add=True)

                return carry

            lax.fori_loop(0, (steps + 1) // 2, pair, 0)
        plsc.subcore_barrier()
        for k in range(ROWS_PER_TILE // 128):
            r = s * ROWS_PER_TILE + k * 128
            pltpu.sync_copy(acc.at[pl.ds(r, 128)], out_hbm.at[pl.ds(r, 128)])

    @pl.when(c == 0)
    def _():
        run(g0_hbm, out0_hbm)

    @pl.when(c == 1)
    def _():
        run(g1_hbm, out1_hbm)


def _make_agg_kernel(steps):
    mesh = plsc.VectorSubcoreMesh(core_axis_name="c", subcore_axis_name="s")
    return pl.kernel(
        _agg_body,
        out_type=(jax.ShapeDtypeStruct((NP, HALF), jnp.float32),
                  jax.ShapeDtypeStruct((NP, HALF), jnp.float32)),
        mesh=mesh,
        scratch_types=[
            pltpu.VMEM((steps, B), jnp.int32),
            pltpu.VMEM((steps, B), jnp.int32),
            pltpu.VMEM((B, HALF), jnp.float32),
            pltpu.VMEM((B, HALF), jnp.float32),
            pltpu.VMEM_SHARED((NP, HALF), jnp.float32),
            pltpu.SemaphoreType.DMA,
            pltpu.SemaphoreType.DMA,
        ],
    )


BLK = 1024
GRID = NP // BLK


def _dis(p0, p1):
    return lax.rsqrt(p0 + p1 + 1.0)


def _tc1_body(x_ref, p0_ref, p1_ref, w_ref, g0_ref, g1_ref):
    dis = _dis(p0_ref[...], p1_ref[...])
    g = jnp.dot(x_ref[...], w_ref[...], preferred_element_type=jnp.float32) * dis
    g0_ref[...] = g[:, :HALF]
    g1_ref[...] = g[:, HALF:]


def _tc2_body(a0_ref, a1_ref, g0_ref, g1_ref, p0_ref, p1_ref, b_ref, w_ref,
              n0_ref, n1_ref):
    dis = _dis(p0_ref[...], p1_ref[...])
    a = jnp.concatenate([a0_ref[...], a1_ref[...]], axis=1)
    g = jnp.concatenate([g0_ref[...], g1_ref[...]], axis=1)
    out1 = jax.nn.relu((a + g) * dis + b_ref[...])
    g2 = jnp.dot(out1, w_ref[...], preferred_element_type=jnp.float32) * dis
    n0_ref[...] = g2[:, :HALF]
    n1_ref[...] = g2[:, HALF:]


def _tc3_body(a0_ref, a1_ref, g0_ref, g1_ref, p0_ref, p1_ref, b_ref, x_ref,
              o_ref):
    dis = _dis(p0_ref[...], p1_ref[...])
    a = jnp.concatenate([a0_ref[...], a1_ref[...]], axis=1)
    g = jnp.concatenate([g0_ref[...], g1_ref[...]], axis=1)
    o_ref[...] = jax.nn.relu((a + g) * dis + b_ref[...] + x_ref[...])


def _row_spec(w):
    return pl.BlockSpec((BLK, w), lambda i: (i, 0))


_FULL_W = pl.BlockSpec((D, D), lambda i: (0, 0))
_FULL_B = pl.BlockSpec((1, D), lambda i: (0, 0))

_tc1 = pl.pallas_call(
    _tc1_body,
    grid=(GRID,),
    in_specs=[_row_spec(D), _row_spec(1), _row_spec(1), _FULL_W],
    out_specs=(_row_spec(HALF), _row_spec(HALF)),
    out_shape=(jax.ShapeDtypeStruct((NP, HALF), jnp.float32),
               jax.ShapeDtypeStruct((NP, HALF), jnp.float32)),
)

_tc2 = pl.pallas_call(
    _tc2_body,
    grid=(GRID,),
    in_specs=[_row_spec(HALF), _row_spec(HALF), _row_spec(HALF), _row_spec(HALF),
              _row_spec(1), _row_spec(1), _FULL_B, _FULL_W],
    out_specs=(_row_spec(HALF), _row_spec(HALF)),
    out_shape=(jax.ShapeDtypeStruct((NP, HALF), jnp.float32),
               jax.ShapeDtypeStruct((NP, HALF), jnp.float32)),
)

_tc3 = pl.pallas_call(
    _tc3_body,
    grid=(GRID,),
    in_specs=[_row_spec(HALF), _row_spec(HALF), _row_spec(HALF), _row_spec(HALF),
              _row_spec(1), _row_spec(1), _FULL_B, _row_spec(D)],
    out_specs=_row_spec(D),
    out_shape=jax.ShapeDtypeStruct((NP, D), jnp.float32),
)


def kernel(x, edge_index, W1, b1, W2, b2):
    E = edge_index.shape[1]
    src = edge_index[0]
    dst = edge_index[1]

    steps = -(-E // (32 * B))
    e_pad = 32 * B * steps
    src_p = jnp.concatenate([src, jnp.zeros((e_pad - E,), jnp.int32)])
    dst_p = jnp.concatenate([dst, jnp.full((e_pad - E,), N, jnp.int32)])
    src_32 = src_p.reshape(32, steps, B)
    dst_32 = dst_p.reshape(32, steps, B)

    x_pad = jnp.pad(x, ((0, NP - x.shape[0]), (0, 0)))
    b1r = b1.reshape(1, D)
    b2r = b2.reshape(1, D)

    p0, p1 = _make_deg_kernel(steps)(dst_32)
    p0 = p0.reshape(NP, 1)
    p1 = p1.reshape(NP, 1)

    agg = _make_agg_kernel(steps)

    g1a, g1b = _tc1(x_pad, p0, p1, W1)
    a1a, a1b = agg(src_32, dst_32, g1a, g1b)
    g2a, g2b = _tc2(a1a, a1b, g1a, g1b, p0, p1, b1r, W2)
    a2a, a2b = agg(src_32, dst_32, g2a, g2b)
    out = _tc3(a2a, a2b, g2a, g2b, p0, p1, b2r, x_pad)
    return out[:N]
